# baseline (speedup 1.0000x reference)
import functools

import jax
import jax.numpy as jnp
from jax import lax
from jax.experimental import pallas as pl
from jax.experimental.pallas import tpu as pltpu
from jax.experimental.pallas import tpu_sc as plsc

_NUM_CODES = 1024
_CODE_DIM = 64
_COMMIT = 0.25


def _vq_tc_body(z_ref, cb_ref, codes_ref, loss_ref, cbt_ref):
    b = pl.program_id(0)
    nb = pl.num_programs(0)
    z_b = z_ref[0]
    cb = cb_ref[...]

    m = jax.lax.dot_general(cb, z_b, (((1,), (0,)), ((), ())),
                            preferred_element_type=jnp.float32)
    x2 = jnp.sum(z_b * z_b, axis=0, keepdims=True)
    e2 = jnp.sum(cb * cb, axis=1, keepdims=True)
    dist = (x2 + e2) - 2.0 * m

    minval = jnp.min(dist, axis=0, keepdims=True)
    iota_c = jax.lax.broadcasted_iota(jnp.int32, dist.shape, 0)
    masked = jnp.where(dist == minval, iota_c, _NUM_CODES)
    codes = jnp.min(masked, axis=0, keepdims=True)
    codes_ref[0] = codes

    partial = jnp.sum(minval)
    prev = jnp.where(b == 0, 0.0, loss_ref[0, 0])
    acc = prev + partial
    scale = _COMMIT / (nb * _CODE_DIM * z_b.shape[1])
    loss_ref[0, 0] = jnp.where(b == nb - 1, acc * scale, acc)

    @pl.when(b == 0)
    def _():
        iota_r = jax.lax.broadcasted_iota(jnp.int32, (_CODE_DIM, _CODE_DIM), 0)
        iota_cc = jax.lax.broadcasted_iota(jnp.int32, (_CODE_DIM, _CODE_DIM), 1)
        eye = (iota_r == iota_cc).astype(jnp.float32)
        cbt_ref[...] = jax.lax.dot_general(
            eye, cb, (((1,), (1,)), ((), ())),
            precision=jax.lax.Precision.HIGHEST,
            preferred_element_type=jnp.float32)


def _tc_call(z, codebook):
    B, C, T = z.shape
    return pl.pallas_call(
        _vq_tc_body,
        grid=(B,),
        in_specs=[
            pl.BlockSpec((1, C, T), lambda b: (b, 0, 0)),
            pl.BlockSpec((_NUM_CODES, C), lambda b: (0, 0)),
        ],
        out_specs=[
            pl.BlockSpec((1, 1, T), lambda b: (b, 0, 0)),
            pl.BlockSpec((1, 1), lambda b: (0, 0), memory_space=pltpu.SMEM),
            pl.BlockSpec((C, _NUM_CODES), lambda b: (0, 0)),
        ],
        out_shape=[
            jax.ShapeDtypeStruct((B, 1, T), jnp.int32),
            jax.ShapeDtypeStruct((1, 1), jnp.float32),
            jax.ShapeDtypeStruct((C, _NUM_CODES), jnp.float32),
        ],
        compiler_params=pltpu.CompilerParams(
            dimension_semantics=("arbitrary",)),
    )(z, codebook)


def _make_sc_gather(B, C, T):
    info = plsc.get_sparse_core_info()
    NC, NS = info.num_cores, info.num_subcores
    NW = NC * NS
    CPW = C // NW
    mesh = plsc.VectorSubcoreMesh(core_axis_name="c", subcore_axis_name="s")

    @functools.partial(
        pl.kernel, mesh=mesh,
        out_type=jax.ShapeDtypeStruct((B * C * T,), jnp.float32),
        scratch_types=[
            pltpu.VMEM((CPW * _NUM_CODES,), jnp.float32),
            pltpu.VMEM((B * T,), jnp.int32),
            pltpu.VMEM((CPW * T,), jnp.float32),
        ],
        compiler_params=pltpu.CompilerParams(needs_layout_passes=False),
    )
    def sc_gather(cbt_hbm, codes_hbm, out_hbm, cbt_v, idx_v, out_v):
        wid = lax.axis_index("s") * NC + lax.axis_index("c")
        c0 = wid * CPW
        pltpu.sync_copy(cbt_hbm.at[pl.ds(c0 * _NUM_CODES, CPW * _NUM_CODES)],
                        cbt_v)
        pltpu.sync_copy(codes_hbm, idx_v)

        def body(b, carry):
            for ci in range(CPW):
                for i in range(T // 16):
                    idx = idx_v[pl.ds(b * T + i * 16, 16)]
                    vals = plsc.load_gather(cbt_v, [idx + (ci * _NUM_CODES)])
                    out_v[pl.ds(ci * T + i * 16, 16)] = vals
            pltpu.sync_copy(out_v,
                            out_hbm.at[pl.ds((b * C + c0) * T, CPW * T)])
            return carry

        lax.fori_loop(0, B, body, 0)

    return sc_gather


def kernel(z, codebook):
    B, C, T = z.shape
    codes3, loss, cbt = _tc_call(z, codebook)
    codes = codes3.reshape(B, T)
    zq_flat = _make_sc_gather(B, C, T)(cbt.reshape(-1), codes.reshape(-1))
    return zq_flat.reshape(B, C, T), codes, loss[0, 0]


# speedup vs baseline: 1.7838x; 1.7838x over previous
import jax
import jax.numpy as jnp
from jax.experimental import pallas as pl
from jax.experimental.pallas import tpu as pltpu

_NUM_CODES = 1024
_CODE_DIM = 64
_COMMIT = 0.25
_BPS = 2


def _vq_body(z_ref, cb_ref, zq_ref, codes_ref, loss_ref):
    s = pl.program_id(0)
    ns = pl.num_programs(0)
    cb = cb_ref[...]
    z_w = jnp.concatenate([z_ref[i] for i in range(_BPS)], axis=1)

    m = jax.lax.dot_general(cb, z_w, (((1,), (0,)), ((), ())),
                            preferred_element_type=jnp.float32)
    x2 = jnp.sum(z_w * z_w, axis=0, keepdims=True)
    e2 = jnp.sum(cb * cb, axis=1, keepdims=True)
    dist = (x2 + e2) - 2.0 * m

    minval = jnp.min(dist, axis=0, keepdims=True)
    iota_c = jax.lax.broadcasted_iota(jnp.int32, dist.shape, 0)
    masked = jnp.where(dist == minval, iota_c, _NUM_CODES)
    codes = jnp.min(masked, axis=0, keepdims=True)

    one_hot = (iota_c == codes).astype(jnp.bfloat16)
    cb_hi = cb.astype(jnp.bfloat16)
    cb_lo = (cb - cb_hi.astype(jnp.float32)).astype(jnp.bfloat16)
    zq_hi = jax.lax.dot_general(cb_hi, one_hot, (((0,), (0,)), ((), ())),
                                preferred_element_type=jnp.float32)
    zq_lo = jax.lax.dot_general(cb_lo, one_hot, (((0,), (0,)), ((), ())),
                                preferred_element_type=jnp.float32)
    zq_w = zq_hi + zq_lo

    T = z_w.shape[1] // _BPS
    for i in range(_BPS):
        z_i = z_ref[i]
        zq_i = zq_w[:, i * T:(i + 1) * T]
        zq_ref[i] = z_i + (zq_i - z_i)
        codes_ref[i] = codes[:, i * T:(i + 1) * T]

    diff = z_w - zq_w
    partial = jnp.sum(diff * diff)
    prev = jnp.where(s == 0, 0.0, loss_ref[0, 0])
    acc = prev + partial
    scale = _COMMIT / (ns * _CODE_DIM * z_w.shape[1])
    loss_ref[0, 0] = jnp.where(s == ns - 1, acc * scale, acc)


def kernel(z, codebook):
    B, C, T = z.shape
    zq, codes3, loss = pl.pallas_call(
        _vq_body,
        grid=(B // _BPS,),
        in_specs=[
            pl.BlockSpec((_BPS, C, T), lambda s: (s, 0, 0)),
            pl.BlockSpec((_NUM_CODES, C), lambda s: (0, 0)),
        ],
        out_specs=[
            pl.BlockSpec((_BPS, C, T), lambda s: (s, 0, 0)),
            pl.BlockSpec((_BPS, 1, T), lambda s: (s, 0, 0)),
            pl.BlockSpec((1, 1), lambda s: (0, 0), memory_space=pltpu.SMEM),
        ],
        out_shape=[
            jax.ShapeDtypeStruct((B, C, T), jnp.float32),
            jax.ShapeDtypeStruct((B, 1, T), jnp.int32),
            jax.ShapeDtypeStruct((1, 1), jnp.float32),
        ],
        compiler_params=pltpu.CompilerParams(
            dimension_semantics=("arbitrary",)),
    )(z, codebook)
    return zq, codes3.reshape(B, T), loss[0, 0]


# speedup vs baseline: 1.8295x; 1.0256x over previous
import jax
import jax.numpy as jnp
from jax.experimental import pallas as pl
from jax.experimental.pallas import tpu as pltpu

_NUM_CODES = 1024
_CODE_DIM = 64
_COMMIT = 0.25
_BPS = 4


def _vq_body(z_ref, cb_ref, zq_ref, codes_ref, loss_ref):
    s = pl.program_id(0)
    ns = pl.num_programs(0)
    cb = cb_ref[...]
    z_w = jnp.concatenate([z_ref[i] for i in range(_BPS)], axis=1)

    m = jax.lax.dot_general(cb, z_w, (((1,), (0,)), ((), ())),
                            preferred_element_type=jnp.float32)
    x2 = jnp.sum(z_w * z_w, axis=0, keepdims=True)
    e2 = jnp.sum(cb * cb, axis=1, keepdims=True)
    dist = (x2 + e2) - 2.0 * m

    minval = jnp.min(dist, axis=0, keepdims=True)
    iota_c = jax.lax.broadcasted_iota(jnp.int32, dist.shape, 0)
    masked = jnp.where(dist == minval, iota_c, _NUM_CODES)
    codes = jnp.min(masked, axis=0, keepdims=True)

    one_hot = (iota_c == codes).astype(jnp.bfloat16)
    cb_hi = cb.astype(jnp.bfloat16)
    cb_lo = (cb - cb_hi.astype(jnp.float32)).astype(jnp.bfloat16)
    zq_hi = jax.lax.dot_general(cb_hi, one_hot, (((0,), (0,)), ((), ())),
                                preferred_element_type=jnp.float32)
    zq_lo = jax.lax.dot_general(cb_lo, one_hot, (((0,), (0,)), ((), ())),
                                preferred_element_type=jnp.float32)
    zq_w = zq_hi + zq_lo

    T = z_w.shape[1] // _BPS
    for i in range(_BPS):
        z_i = z_ref[i]
        zq_i = zq_w[:, i * T:(i + 1) * T]
        zq_ref[i] = z_i + (zq_i - z_i)
        codes_ref[i] = codes[:, i * T:(i + 1) * T]

    diff = z_w - zq_w
    partial = jnp.sum(diff * diff)
    prev = jnp.where(s == 0, 0.0, loss_ref[0, 0])
    acc = prev + partial
    scale = _COMMIT / (ns * _CODE_DIM * z_w.shape[1])
    loss_ref[0, 0] = jnp.where(s == ns - 1, acc * scale, acc)


def kernel(z, codebook):
    B, C, T = z.shape
    zq, codes3, loss = pl.pallas_call(
        _vq_body,
        grid=(B // _BPS,),
        in_specs=[
            pl.BlockSpec((_BPS, C, T), lambda s: (s, 0, 0)),
            pl.BlockSpec((_NUM_CODES, C), lambda s: (0, 0)),
        ],
        out_specs=[
            pl.BlockSpec((_BPS, C, T), lambda s: (s, 0, 0)),
            pl.BlockSpec((_BPS, 1, T), lambda s: (s, 0, 0)),
            pl.BlockSpec((1, 1), lambda s: (0, 0), memory_space=pltpu.SMEM),
        ],
        out_shape=[
            jax.ShapeDtypeStruct((B, C, T), jnp.float32),
            jax.ShapeDtypeStruct((B, 1, T), jnp.int32),
            jax.ShapeDtypeStruct((1, 1), jnp.float32),
        ],
        compiler_params=pltpu.CompilerParams(
            dimension_semantics=("arbitrary",)),
    )(z, codebook)
    return zq, codes3.reshape(B, T), loss[0, 0]


# speedup vs baseline: 1.8924x; 1.0344x over previous
import jax
import jax.numpy as jnp
from jax.experimental import pallas as pl
from jax.experimental.pallas import tpu as pltpu

_NUM_CODES = 1024
_CODE_DIM = 64
_COMMIT = 0.25
_BPS = 4


def _vq_body(z_ref, cb_ref, zq_ref, codes_ref, loss_ref):
    s = pl.program_id(0)
    ns = pl.num_programs(0)
    cb = cb_ref[...]
    z_w = jnp.concatenate([z_ref[i] for i in range(_BPS)], axis=1)

    mneg2 = jax.lax.dot_general(-2.0 * cb, z_w, (((1,), (0,)), ((), ())),
                                preferred_element_type=jnp.float32)
    x2 = jnp.sum(z_w * z_w, axis=0, keepdims=True)
    e2 = jnp.sum(cb * cb, axis=1, keepdims=True)
    dist = (x2 + e2) + mneg2

    minval = jnp.min(dist, axis=0, keepdims=True)
    iota_c = jax.lax.broadcasted_iota(jnp.int32, dist.shape, 0)
    masked = jnp.where(dist == minval, iota_c, _NUM_CODES)
    codes = jnp.min(masked, axis=0, keepdims=True)

    one_hot = (iota_c == codes).astype(jnp.bfloat16)
    cb_hi = cb.astype(jnp.bfloat16)
    cb_lo = (cb - cb_hi.astype(jnp.float32)).astype(jnp.bfloat16)
    zq_hi = jax.lax.dot_general(cb_hi, one_hot, (((0,), (0,)), ((), ())),
                                preferred_element_type=jnp.float32)
    zq_lo = jax.lax.dot_general(cb_lo, one_hot, (((0,), (0,)), ((), ())),
                                preferred_element_type=jnp.float32)
    zq_w = zq_hi + zq_lo

    T = z_w.shape[1] // _BPS
    for i in range(_BPS):
        z_i = z_ref[i]
        zq_i = zq_w[:, i * T:(i + 1) * T]
        zq_ref[i] = z_i + (zq_i - z_i)
        codes_ref[i] = codes[:, i * T:(i + 1) * T]

    diff = z_w - zq_w
    partial = jnp.sum(diff * diff)
    prev = jnp.where(s == 0, 0.0, loss_ref[0, 0])
    acc = prev + partial
    scale = _COMMIT / (ns * _CODE_DIM * z_w.shape[1])
    loss_ref[0, 0] = jnp.where(s == ns - 1, acc * scale, acc)


def kernel(z, codebook):
    B, C, T = z.shape
    zq, codes3, loss = pl.pallas_call(
        _vq_body,
        grid=(B // _BPS,),
        in_specs=[
            pl.BlockSpec((_BPS, C, T), lambda s: (s, 0, 0)),
            pl.BlockSpec((_NUM_CODES, C), lambda s: (0, 0)),
        ],
        out_specs=[
            pl.BlockSpec((_BPS, C, T), lambda s: (s, 0, 0)),
            pl.BlockSpec((_BPS, 1, T), lambda s: (s, 0, 0)),
            pl.BlockSpec((1, 1), lambda s: (0, 0), memory_space=pltpu.SMEM),
        ],
        out_shape=[
            jax.ShapeDtypeStruct((B, C, T), jnp.float32),
            jax.ShapeDtypeStruct((B, 1, T), jnp.int32),
            jax.ShapeDtypeStruct((1, 1), jnp.float32),
        ],
        compiler_params=pltpu.CompilerParams(
            dimension_semantics=("arbitrary",)),
    )(z, codebook)
    return zq, codes3.reshape(B, T), loss[0, 0]


# speedup vs baseline: 1.9054x; 1.0069x over previous
import jax
import jax.numpy as jnp
from jax.experimental import pallas as pl
from jax.experimental.pallas import tpu as pltpu

_NUM_CODES = 1024
_CODE_DIM = 64
_COMMIT = 0.25
_BPS = 8


def _vq_body(z_ref, cb_ref, zq_ref, codes_ref, loss_ref):
    s = pl.program_id(0)
    ns = pl.num_programs(0)
    cb = cb_ref[...]
    z_w = jnp.concatenate([z_ref[i] for i in range(_BPS)], axis=1)

    mneg2 = jax.lax.dot_general(-2.0 * cb, z_w, (((1,), (0,)), ((), ())),
                                preferred_element_type=jnp.float32)
    x2 = jnp.sum(z_w * z_w, axis=0, keepdims=True)
    e2 = jnp.sum(cb * cb, axis=1, keepdims=True)
    dist = (x2 + e2) + mneg2

    minval = jnp.min(dist, axis=0, keepdims=True)
    iota_c = jax.lax.broadcasted_iota(jnp.int32, dist.shape, 0)
    masked = jnp.where(dist == minval, iota_c, _NUM_CODES)
    codes = jnp.min(masked, axis=0, keepdims=True)

    one_hot = (iota_c == codes).astype(jnp.bfloat16)
    cb_hi = cb.astype(jnp.bfloat16)
    cb_lo = (cb - cb_hi.astype(jnp.float32)).astype(jnp.bfloat16)
    zq_hi = jax.lax.dot_general(cb_hi, one_hot, (((0,), (0,)), ((), ())),
                                preferred_element_type=jnp.float32)
    zq_lo = jax.lax.dot_general(cb_lo, one_hot, (((0,), (0,)), ((), ())),
                                preferred_element_type=jnp.float32)
    zq_w = zq_hi + zq_lo

    T = z_w.shape[1] // _BPS
    for i in range(_BPS):
        z_i = z_ref[i]
        zq_i = zq_w[:, i * T:(i + 1) * T]
        zq_ref[i] = z_i + (zq_i - z_i)
        codes_ref[i] = codes[:, i * T:(i + 1) * T]

    diff = z_w - zq_w
    partial = jnp.sum(diff * diff)
    prev = jnp.where(s == 0, 0.0, loss_ref[0, 0])
    acc = prev + partial
    scale = _COMMIT / (ns * _CODE_DIM * z_w.shape[1])
    loss_ref[0, 0] = jnp.where(s == ns - 1, acc * scale, acc)


def kernel(z, codebook):
    B, C, T = z.shape
    zq, codes3, loss = pl.pallas_call(
        _vq_body,
        grid=(B // _BPS,),
        in_specs=[
            pl.BlockSpec((_BPS, C, T), lambda s: (s, 0, 0)),
            pl.BlockSpec((_NUM_CODES, C), lambda s: (0, 0)),
        ],
        out_specs=[
            pl.BlockSpec((_BPS, C, T), lambda s: (s, 0, 0)),
            pl.BlockSpec((_BPS, 1, T), lambda s: (s, 0, 0)),
            pl.BlockSpec((1, 1), lambda s: (0, 0), memory_space=pltpu.SMEM),
        ],
        out_shape=[
            jax.ShapeDtypeStruct((B, C, T), jnp.float32),
            jax.ShapeDtypeStruct((B, 1, T), jnp.int32),
            jax.ShapeDtypeStruct((1, 1), jnp.float32),
        ],
        compiler_params=pltpu.CompilerParams(
            dimension_semantics=("arbitrary",),
            vmem_limit_bytes=120 * 1024 * 1024),
    )(z, codebook)
    return zq, codes3.reshape(B, T), loss[0, 0]
